# baseline (speedup 1.0000x reference)
import functools
import math

import jax
import jax.numpy as jnp
from jax import lax
from jax.experimental import pallas as pl
from jax.experimental.pallas import tpu as pltpu
from jax.experimental.pallas import tpu_sc as plsc

DIM = 64
SEQ_LEN = 200
BATCH = 4096
NC = 2
NS = 16
NW = NC * NS
CHUNK = 128
SCALE = math.sqrt(DIM)


def _make_sc_gather(n_flat):
    per_w = n_flat // NW
    n_chunks = per_w // CHUNK
    mesh = plsc.VectorSubcoreMesh(core_axis_name="c", subcore_axis_name="s")

    @functools.partial(
        pl.kernel,
        mesh=mesh,
        out_type=jax.ShapeDtypeStruct((n_flat, DIM), jnp.float32),
        compiler_params=pltpu.CompilerParams(use_tc_tiling_on_sc=False),
        scratch_types=[
            pltpu.VMEM((CHUNK,), jnp.int32),
            pltpu.VMEM((CHUNK, DIM), jnp.float32),
            pltpu.VMEM((SEQ_LEN, DIM), jnp.float32),
            pltpu.SemaphoreType.DMA,
        ],
    )
    def body(idx_hbm, table_hbm, pe_hbm, out_hbm, idx_v, rows_v, pe_v, sem):
        wid = lax.axis_index("c") * NS + lax.axis_index("s")
        w_base = wid * per_w
        pltpu.sync_copy(pe_hbm.at[pl.ds(0, SEQ_LEN)], pe_v)

        def chunk_body(c, carry):
            base = w_base + c * CHUNK
            s = base // BATCH
            pltpu.sync_copy(idx_hbm.at[pl.ds(base, CHUNK)], idx_v)
            pltpu.async_copy(table_hbm.at[idx_v], rows_v, sem).wait()
            pe0 = pe_v[s, pl.ds(0, 16)]
            pe1 = pe_v[s, pl.ds(16, 16)]
            pe2 = pe_v[s, pl.ds(32, 16)]
            pe3 = pe_v[s, pl.ds(48, 16)]

            def row_body(r, rcarry):
                rows_v[r, pl.ds(0, 16)] = rows_v[r, pl.ds(0, 16)] * SCALE + pe0
                rows_v[r, pl.ds(16, 16)] = rows_v[r, pl.ds(16, 16)] * SCALE + pe1
                rows_v[r, pl.ds(32, 16)] = rows_v[r, pl.ds(32, 16)] * SCALE + pe2
                rows_v[r, pl.ds(48, 16)] = rows_v[r, pl.ds(48, 16)] * SCALE + pe3
                return rcarry

            lax.fori_loop(0, CHUNK, row_body, 0)
            pltpu.sync_copy(rows_v, out_hbm.at[pl.ds(base, CHUNK)])
            return carry

        lax.fori_loop(0, n_chunks, chunk_body, 0)

    return body


def kernel(x, emb_weight, positional_encodings):
    seq, batch = x.shape
    idx_flat = x.reshape(-1)
    pe2d = positional_encodings.reshape(positional_encodings.shape[0], DIM)
    out_flat = _make_sc_gather(seq * batch)(idx_flat, emb_weight, pe2d)
    return out_flat.reshape(seq, batch, DIM)


# speedup vs baseline: 1.2542x; 1.2542x over previous
import functools
import math

import jax
import jax.numpy as jnp
from jax import lax
from jax.experimental import pallas as pl
from jax.experimental.pallas import tpu as pltpu
from jax.experimental.pallas import tpu_sc as plsc

DIM = 64
SEQ_LEN = 200
BATCH = 4096
NC = 2
NS = 16
NW = NC * NS
CHUNK = 128
SCALE = math.sqrt(DIM)


def _make_sc_gather(n_flat):
    per_w = n_flat // NW
    n_chunks = per_w // CHUNK
    assert n_chunks % 2 == 0
    mesh = plsc.VectorSubcoreMesh(core_axis_name="c", subcore_axis_name="s")

    @functools.partial(
        pl.kernel,
        mesh=mesh,
        out_type=jax.ShapeDtypeStruct((n_flat, DIM), jnp.float32),
        compiler_params=pltpu.CompilerParams(use_tc_tiling_on_sc=False),
        scratch_types=[
            pltpu.VMEM((n_chunks, CHUNK), jnp.int32),
            pltpu.VMEM((SEQ_LEN, DIM), jnp.float32),
            pltpu.VMEM((CHUNK, DIM), jnp.float32),
            pltpu.VMEM((CHUNK, DIM), jnp.float32),
            pltpu.SemaphoreType.DMA,
            pltpu.SemaphoreType.DMA,
            pltpu.SemaphoreType.DMA,
            pltpu.SemaphoreType.DMA,
        ],
    )
    def body(idx_hbm, table_hbm, pe_hbm, out_hbm, idx_v, pe_v,
             rows0, rows1, sg0, sg1, sw0, sw1):
        wid = lax.axis_index("c") * NS + lax.axis_index("s")
        w_row = wid * n_chunks
        w_base = wid * per_w
        rows = (rows0, rows1)
        sg = (sg0, sg1)
        sw = (sw0, sw1)
        pltpu.sync_copy(idx_hbm.at[pl.ds(w_row, n_chunks)], idx_v)
        pltpu.sync_copy(pe_hbm.at[pl.ds(0, SEQ_LEN)], pe_v)

        def gather(c, b):
            pltpu.async_copy(table_hbm.at[idx_v.at[c]], rows[b], sg[b])

        def wait_g(b):
            pltpu.make_async_copy(table_hbm.at[idx_v.at[0]], rows[b], sg[b]).wait()

        def wait_w(b):
            pltpu.make_async_copy(rows[b], out_hbm.at[pl.ds(0, CHUNK)], sw[b]).wait()

        gather(0, 0)

        def outer(c2, carry):
            for b in (0, 1):
                c = c2 * 2 + b
                q = 1 - b
                @pl.when(c + 1 < n_chunks)
                def _():
                    @pl.when(c > 0)
                    def _():
                        wait_w(q)
                    gather(c + 1, q)

                wait_g(b)
                base = w_base + c * CHUNK
                s = base // BATCH
                pe0 = pe_v[s, pl.ds(0, 16)]
                pe1 = pe_v[s, pl.ds(16, 16)]
                pe2 = pe_v[s, pl.ds(32, 16)]
                pe3 = pe_v[s, pl.ds(48, 16)]
                rv = rows[b]

                def row_body(r, rcarry):
                    rv[r, pl.ds(0, 16)] = rv[r, pl.ds(0, 16)] * SCALE + pe0
                    rv[r, pl.ds(16, 16)] = rv[r, pl.ds(16, 16)] * SCALE + pe1
                    rv[r, pl.ds(32, 16)] = rv[r, pl.ds(32, 16)] * SCALE + pe2
                    rv[r, pl.ds(48, 16)] = rv[r, pl.ds(48, 16)] * SCALE + pe3
                    return rcarry

                lax.fori_loop(0, CHUNK, row_body, 0, unroll=4)
                pltpu.async_copy(rv, out_hbm.at[pl.ds(base, CHUNK)], sw[b])
            return carry

        lax.fori_loop(0, n_chunks // 2, outer, 0)
        wait_w(0)
        wait_w(1)

    return body


def kernel(x, emb_weight, positional_encodings):
    seq, batch = x.shape
    idx2d = x.reshape(-1, CHUNK)
    pe2d = positional_encodings.reshape(positional_encodings.shape[0], DIM)
    out_flat = _make_sc_gather(seq * batch)(idx2d, emb_weight, pe2d)
    return out_flat.reshape(seq, batch, DIM)
